# Initial kernel scaffold; baseline (speedup 1.0000x reference)
import jax
import jax.numpy as jnp
from jax.experimental import pallas as pl


def kernel(x, edge_index, W1, b1, W2, b2, W3, b3):
    raise NotImplementedError("write your pallas kernel here")


# speedup vs baseline: 44.5868x; 44.5868x over previous
import functools

import jax
import jax.numpy as jnp
from jax import lax
from jax.experimental import pallas as pl
from jax.experimental.pallas import tpu as pltpu
from jax.experimental.pallas import tpu_sc as plsc

N = 100000
E = 3200000
D = 16

N_PAD = 100352
E_PAD = 3211264
R = E_PAD // 128
NC, NS = 2, 16
ROWS_PER_TILE = R // (NC * NS)
STRIPE = N_PAD // NS
G = 8
MACROS = ROWS_PER_TILE // G
BLK = 1024
GRID = N_PAD // BLK

_MESH = plsc.VectorSubcoreMesh(
    core_axis_name="c", subcore_axis_name="s", num_cores=NC, num_subcores=NS)


def _wid():
    return lax.axis_index("s") * NC + lax.axis_index("c")


@functools.partial(
    pl.kernel,
    out_type=jax.ShapeDtypeStruct((NC, N_PAD), jnp.float32),
    mesh=_MESH,
    compiler_params=pltpu.CompilerParams(use_tc_tiling_on_sc=False),
    scratch_types=[
        pltpu.VMEM_SHARED((N_PAD,), jnp.float32),
        pltpu.VMEM((784,), jnp.float32),
        pltpu.VMEM((128,), jnp.float32),
        pltpu.VMEM((G, 128), jnp.int32),
    ],
)
def _deg_kernel(dst_hbm, z1_hbm, ones_hbm, out_hbm, acc, zb, ones_v, dbuf):
    c = lax.axis_index("c")
    s = lax.axis_index("s")
    pltpu.sync_copy(z1_hbm, zb)
    pltpu.sync_copy(ones_hbm, ones_v)
    r0 = s * STRIPE
    for j in range(STRIPE // 784):
        pltpu.sync_copy(zb, acc.at[pl.ds(r0 + j * 784, 784)])
    plsc.subcore_barrier()
    rows0 = _wid() * ROWS_PER_TILE

    def step(g, carry):
        base = rows0 + g * G
        pltpu.sync_copy(dst_hbm.at[pl.ds(base, G)], dbuf)
        for j in range(G):
            pltpu.sync_copy(ones_v, acc.at[dbuf.at[j]], add=True)
        return carry

    lax.fori_loop(0, MACROS, step, 0)
    plsc.subcore_barrier()
    pltpu.sync_copy(acc.at[pl.ds(r0, STRIPE)], out_hbm.at[c, pl.ds(r0, STRIPE)])


@functools.partial(
    pl.kernel,
    out_type=jax.ShapeDtypeStruct((NC, N_PAD, D), jnp.float32),
    mesh=_MESH,
    compiler_params=pltpu.CompilerParams(use_tc_tiling_on_sc=False),
    scratch_types=[
        pltpu.VMEM_SHARED((N_PAD, D), jnp.float32),
        pltpu.VMEM((392, D), jnp.float32),
        pltpu.VMEM((G, 128), jnp.int32),
        pltpu.VMEM((G, 128), jnp.int32),
        pltpu.VMEM((G, 128, D), jnp.float32),
        pltpu.SemaphoreType.DMA,
    ],
)
def _prop_kernel(table_hbm, src_hbm, dst_hbm, z16_hbm, out_hbm,
                 acc, zb, sbuf, dbuf, rows, gsem):
    c = lax.axis_index("c")
    s = lax.axis_index("s")
    pltpu.sync_copy(z16_hbm, zb)
    r0 = s * STRIPE
    for j in range(STRIPE // 392):
        pltpu.sync_copy(zb, acc.at[pl.ds(r0 + j * 392, 392)])
    plsc.subcore_barrier()
    rows0 = _wid() * ROWS_PER_TILE

    def step(g, carry):
        base = rows0 + g * G
        pltpu.sync_copy(src_hbm.at[pl.ds(base, G)], sbuf)
        pltpu.sync_copy(dst_hbm.at[pl.ds(base, G)], dbuf)
        descs = [pltpu.async_copy(table_hbm.at[sbuf.at[j]], rows.at[j], gsem)
                 for j in range(G)]
        for d in descs:
            d.wait()
        for j in range(G):
            pltpu.sync_copy(rows.at[j], acc.at[dbuf.at[j]], add=True)
        return carry

    lax.fori_loop(0, MACROS, step, 0)
    plsc.subcore_barrier()
    pltpu.sync_copy(acc.at[pl.ds(r0, STRIPE)], out_hbm.at[c, pl.ds(r0, STRIPE)])


def _row_spec(width):
    return pl.BlockSpec((BLK, width), lambda i: (i, 0))


def _part_spec(width):
    return pl.BlockSpec((NC, BLK, width), lambda i: (0, i, 0))


def _full_spec(a, b):
    return pl.BlockSpec((a, b), lambda i: (0, 0))


def _tc1_body(deg_ref, x_ref, w1_ref, dis_ref, h1s_ref):
    d = 1.0 + deg_ref[0] + deg_ref[1]
    dis = lax.rsqrt(d)
    dis_ref[...] = dis
    h1s_ref[...] = dis * jnp.dot(x_ref[...], w1_ref[...],
                                 preferred_element_type=jnp.float32)


def _tc_mid_body(acc_ref, hs_ref, dis_ref, w_ref, b_ref, out_ref, *,
                 final_scale):
    dis = dis_ref[...]
    tot = acc_ref[0] + acc_ref[1] + hs_ref[...]
    o = jax.nn.relu(dis * tot + b_ref[...])
    if final_scale:
        out_ref[...] = dis * o
    else:
        out_ref[...] = dis * jnp.dot(o, w_ref[...],
                                     preferred_element_type=jnp.float32)


def _tc4_body(acc_ref, t3s_ref, dis_ref, w3_ref, b3_ref, out_ref):
    dis = dis_ref[...]
    p = dis * (acc_ref[0] + acc_ref[1] + t3s_ref[...])
    z = jnp.dot(p, w3_ref[...], preferred_element_type=jnp.float32)
    out_ref[...] = jax.nn.sigmoid(z + b3_ref[...])


def kernel(x, edge_index, W1, b1, W2, b2, W3, b3):
    f32 = jnp.float32
    src = edge_index[0].astype(jnp.int32)
    dst = edge_index[1].astype(jnp.int32)
    pad = E_PAD - E
    padv = jnp.full((pad,), N, jnp.int32)
    src2d = jnp.concatenate([src, padv]).reshape(R, 128)
    dst2d = jnp.concatenate([dst, padv]).reshape(R, 128)
    x_pad = jnp.pad(x.astype(f32), ((0, N_PAD - N), (0, 0)))

    z1 = jnp.zeros((784,), f32)
    ones1 = jnp.ones((128,), f32)
    z16 = jnp.zeros((392, D), f32)

    deg_parts = _deg_kernel(dst2d, z1, ones1)
    deg3 = deg_parts.reshape(NC, N_PAD, 1)

    w1 = W1.astype(f32)
    w2 = W2.astype(f32)
    w3 = W3.astype(f32)
    b1r = b1.astype(f32).reshape(1, D)
    b2r = b2.astype(f32).reshape(1, D)
    b3r = b3.astype(f32).reshape(1, 1)

    dis, h1s = pl.pallas_call(
        _tc1_body,
        grid=(GRID,),
        in_specs=[_part_spec(1), _row_spec(D), _full_spec(D, D)],
        out_specs=[_row_spec(1), _row_spec(D)],
        out_shape=[jax.ShapeDtypeStruct((N_PAD, 1), f32),
                   jax.ShapeDtypeStruct((N_PAD, D), f32)],
    )(deg3, x_pad, w1)

    acc1 = _prop_kernel(h1s, src2d, dst2d, z16)

    h2s = pl.pallas_call(
        functools.partial(_tc_mid_body, final_scale=False),
        grid=(GRID,),
        in_specs=[_part_spec(D), _row_spec(D), _row_spec(1),
                  _full_spec(D, D), _full_spec(1, D)],
        out_specs=_row_spec(D),
        out_shape=jax.ShapeDtypeStruct((N_PAD, D), f32),
    )(acc1, h1s, dis, w2, b1r)

    acc2 = _prop_kernel(h2s, src2d, dst2d, z16)

    t3s = pl.pallas_call(
        functools.partial(_tc_mid_body, final_scale=True),
        grid=(GRID,),
        in_specs=[_part_spec(D), _row_spec(D), _row_spec(1),
                  _full_spec(D, D), _full_spec(1, D)],
        out_specs=_row_spec(D),
        out_shape=jax.ShapeDtypeStruct((N_PAD, D), f32),
    )(acc2, h2s, dis, w2, b2r)

    acc3 = _prop_kernel(t3s, src2d, dst2d, z16)

    out = pl.pallas_call(
        _tc4_body,
        grid=(GRID,),
        in_specs=[_part_spec(D), _row_spec(D), _row_spec(1),
                  _full_spec(D, 1), _full_spec(1, 1)],
        out_specs=_row_spec(1),
        out_shape=jax.ShapeDtypeStruct((N_PAD, 1), f32),
    )(acc3, t3s, dis, w3, b3r)

    return out[:N]


# speedup vs baseline: 54.3224x; 1.2184x over previous
import functools

import jax
import jax.numpy as jnp
from jax import lax
from jax.experimental import pallas as pl
from jax.experimental.pallas import tpu as pltpu
from jax.experimental.pallas import tpu_sc as plsc

N = 100000
E = 3200000
D = 16

N_PAD = 100352
E_PAD = 3211264
R = E_PAD // 128
NC, NS = 2, 16
ROWS_PER_TILE = R // (NC * NS)
STRIPE = N_PAD // NS
GD = 8
MD = ROWS_PER_TILE // GD
GP = 4
MP = ROWS_PER_TILE // GP
BLK = 1024
GRID = N_PAD // BLK

_MESH = plsc.VectorSubcoreMesh(
    core_axis_name="c", subcore_axis_name="s", num_cores=NC, num_subcores=NS)
_SC_PARAMS = pltpu.CompilerParams(use_tc_tiling_on_sc=False)


def _wid():
    return lax.axis_index("s") * NC + lax.axis_index("c")


@functools.partial(
    pl.kernel,
    out_type=jax.ShapeDtypeStruct((NC, N_PAD), jnp.float32),
    mesh=_MESH,
    compiler_params=_SC_PARAMS,
    scratch_types=[
        pltpu.VMEM_SHARED((N_PAD,), jnp.float32),
        pltpu.VMEM((128,), jnp.float32),
        pltpu.VMEM((2, GD, 2, 128), jnp.int32),
        pltpu.SemaphoreType.DMA,
        pltpu.SemaphoreType.DMA,
    ],
)
def _deg_kernel(idx_hbm, z1_hbm, ones_hbm, out_hbm, acc, ones_v, ibuf,
                isem, ssem):
    c = lax.axis_index("c")
    s = lax.axis_index("s")
    pltpu.sync_copy(ones_hbm, ones_v)
    r0 = s * STRIPE
    for j in range(STRIPE // 784):
        pltpu.sync_copy(z1_hbm, acc.at[pl.ds(r0 + j * 784, 784)])
    plsc.subcore_barrier()
    rows0 = _wid() * ROWS_PER_TILE
    last = rows0 + (MD - 1) * GD

    def idx_fire(base, b):
        pltpu.async_copy(idx_hbm.at[pl.ds(base, GD)], ibuf.at[b], isem)

    def idx_wait(b):
        pltpu.make_async_copy(idx_hbm.at[pl.ds(0, GD)], ibuf.at[b], isem).wait()

    idx_fire(rows0, 0)
    idx_fire(rows0 + GD, 1)

    def outer(k, carry):
        g0 = k * 2
        for u in range(2):
            b = u
            g = g0 + u
            idx_wait(b)
            for j in range(GD):
                pltpu.async_copy(ones_v, acc.at[ibuf.at[b, j, 1]], ssem,
                                 add=True)
            for j in range(GD):
                pltpu.make_async_copy(
                    ones_v, acc.at[ibuf.at[b, j, 1]], ssem).wait()
            nbase = lax.min(rows0 + (g + 2) * GD, last)
            idx_fire(nbase, b)
        return carry

    lax.fori_loop(0, MD // 2, outer, 0)
    idx_wait(0)
    idx_wait(1)
    plsc.subcore_barrier()
    pltpu.sync_copy(acc.at[pl.ds(r0, STRIPE)], out_hbm.at[c, pl.ds(r0, STRIPE)])


@functools.partial(
    pl.kernel,
    out_type=jax.ShapeDtypeStruct((NC, N_PAD, D), jnp.float32),
    mesh=_MESH,
    compiler_params=_SC_PARAMS,
    scratch_types=[
        pltpu.VMEM_SHARED((N_PAD, D), jnp.float32),
        pltpu.VMEM((2, GP, 2, 128), jnp.int32),
        pltpu.VMEM((2, GP, 128, D), jnp.float32),
        pltpu.SemaphoreType.DMA,
        pltpu.SemaphoreType.DMA,
        pltpu.SemaphoreType.DMA,
        pltpu.SemaphoreType.DMA,
        pltpu.SemaphoreType.DMA,
    ],
)
def _prop_kernel(table_hbm, idx_hbm, z16_hbm, out_hbm,
                 acc, ibuf, rows, isem, gsem0, gsem1, ssem0, ssem1):
    c = lax.axis_index("c")
    s = lax.axis_index("s")
    gsem = (gsem0, gsem1)
    ssem = (ssem0, ssem1)
    r0 = s * STRIPE
    for j in range(STRIPE // 392):
        pltpu.sync_copy(z16_hbm, acc.at[pl.ds(r0 + j * 392, 392)])
    plsc.subcore_barrier()
    rows0 = _wid() * ROWS_PER_TILE
    last = rows0 + (MP - 1) * GP

    def idx_fire(base, b):
        pltpu.async_copy(idx_hbm.at[pl.ds(base, GP)], ibuf.at[b], isem)

    def idx_wait(b):
        pltpu.make_async_copy(idx_hbm.at[pl.ds(0, GP)], ibuf.at[b], isem).wait()

    def gather_fire(b):
        for j in range(GP):
            pltpu.async_copy(table_hbm.at[ibuf.at[b, j, 0]], rows.at[b, j],
                             gsem[b])

    def gather_drain(b):
        for j in range(GP):
            pltpu.make_async_copy(table_hbm.at[ibuf.at[b, j, 0]],
                                  rows.at[b, j], gsem[b]).wait()

    def scatter_fire(b):
        for j in range(GP):
            pltpu.async_copy(rows.at[b, j], acc.at[ibuf.at[b, j, 1]], ssem[b],
                             add=True)

    def scatter_drain(b):
        for j in range(GP):
            pltpu.make_async_copy(rows.at[b, j], acc.at[ibuf.at[b, j, 1]],
                                  ssem[b]).wait()

    idx_fire(rows0, 0)
    idx_wait(0)
    gather_fire(0)
    idx_fire(rows0 + GP, 1)

    def outer(k, carry):
        g0 = k * 2
        for u in range(2):
            b = u
            bn = 1 - u
            g = g0 + u
            idx_wait(bn)
            gather_drain(b)
            gather_fire(bn)
            scatter_fire(b)
            scatter_drain(b)
            nbase = lax.min(rows0 + (g + 2) * GP, last)
            idx_fire(nbase, b)
        return carry

    lax.fori_loop(0, MP // 2, outer, 0)
    idx_wait(0)
    gather_drain(0)
    plsc.subcore_barrier()
    pltpu.sync_copy(acc.at[pl.ds(r0, STRIPE)], out_hbm.at[c, pl.ds(r0, STRIPE)])


def _row_spec(width):
    return pl.BlockSpec((BLK, width), lambda i: (i, 0))


def _part_spec(width):
    return pl.BlockSpec((NC, BLK, width), lambda i: (0, i, 0))


def _full_spec(a, b):
    return pl.BlockSpec((a, b), lambda i: (0, 0))


def _tc1_body(deg_ref, x_ref, w1_ref, dis_ref, h1s_ref):
    d = 1.0 + deg_ref[0] + deg_ref[1]
    dis = lax.rsqrt(d)
    dis_ref[...] = dis
    h1s_ref[...] = dis * jnp.dot(x_ref[...], w1_ref[...],
                                 preferred_element_type=jnp.float32)


def _tc_mid_body(acc_ref, hs_ref, dis_ref, w_ref, b_ref, out_ref, *,
                 final_scale):
    dis = dis_ref[...]
    tot = acc_ref[0] + acc_ref[1] + hs_ref[...]
    o = jax.nn.relu(dis * tot + b_ref[...])
    if final_scale:
        out_ref[...] = dis * o
    else:
        out_ref[...] = dis * jnp.dot(o, w_ref[...],
                                     preferred_element_type=jnp.float32)


def _tc4_body(acc_ref, t3s_ref, dis_ref, w3_ref, b3_ref, out_ref):
    dis = dis_ref[...]
    p = dis * (acc_ref[0] + acc_ref[1] + t3s_ref[...])
    z = jnp.dot(p, w3_ref[...], preferred_element_type=jnp.float32)
    out_ref[...] = jax.nn.sigmoid(z + b3_ref[...])


def kernel(x, edge_index, W1, b1, W2, b2, W3, b3):
    f32 = jnp.float32
    ei = edge_index.astype(jnp.int32)
    pad = E_PAD - E
    padv = jnp.full((2, pad), N, jnp.int32)
    idx2 = (jnp.concatenate([ei, padv], axis=1)
            .reshape(2, R, 128).transpose(1, 0, 2))
    x_pad = jnp.pad(x.astype(f32), ((0, N_PAD - N), (0, 0)))

    z1 = jnp.zeros((784,), f32)
    ones1 = jnp.ones((128,), f32)
    z16 = jnp.zeros((392, D), f32)

    deg_parts = _deg_kernel(idx2, z1, ones1)
    deg3 = deg_parts.reshape(NC, N_PAD, 1)

    w1 = W1.astype(f32)
    w2 = W2.astype(f32)
    w3 = W3.astype(f32)
    b1r = b1.astype(f32).reshape(1, D)
    b2r = b2.astype(f32).reshape(1, D)
    b3r = b3.astype(f32).reshape(1, 1)

    dis, h1s = pl.pallas_call(
        _tc1_body,
        grid=(GRID,),
        in_specs=[_part_spec(1), _row_spec(D), _full_spec(D, D)],
        out_specs=[_row_spec(1), _row_spec(D)],
        out_shape=[jax.ShapeDtypeStruct((N_PAD, 1), f32),
                   jax.ShapeDtypeStruct((N_PAD, D), f32)],
    )(deg3, x_pad, w1)

    acc1 = _prop_kernel(h1s, idx2, z16)

    h2s = pl.pallas_call(
        functools.partial(_tc_mid_body, final_scale=False),
        grid=(GRID,),
        in_specs=[_part_spec(D), _row_spec(D), _row_spec(1),
                  _full_spec(D, D), _full_spec(1, D)],
        out_specs=_row_spec(D),
        out_shape=jax.ShapeDtypeStruct((N_PAD, D), f32),
    )(acc1, h1s, dis, w2, b1r)

    acc2 = _prop_kernel(h2s, idx2, z16)

    t3s = pl.pallas_call(
        functools.partial(_tc_mid_body, final_scale=True),
        grid=(GRID,),
        in_specs=[_part_spec(D), _row_spec(D), _row_spec(1),
                  _full_spec(D, D), _full_spec(1, D)],
        out_specs=_row_spec(D),
        out_shape=jax.ShapeDtypeStruct((N_PAD, D), f32),
    )(acc2, h2s, dis, w2, b2r)

    acc3 = _prop_kernel(t3s, idx2, z16)

    out = pl.pallas_call(
        _tc4_body,
        grid=(GRID,),
        in_specs=[_part_spec(D), _row_spec(D), _row_spec(1),
                  _full_spec(D, 1), _full_spec(1, 1)],
        out_specs=_row_spec(1),
        out_shape=jax.ShapeDtypeStruct((N_PAD, 1), f32),
    )(acc3, t3s, dis, w3, b3r)

    return out[:N]


# speedup vs baseline: 72.8022x; 1.3402x over previous
import functools

import jax
import jax.numpy as jnp
from jax import lax
from jax.experimental import pallas as pl
from jax.experimental.pallas import tpu as pltpu
from jax.experimental.pallas import tpu_sc as plsc

N = 100000
E = 3200000
D = 16

N_PAD = 100352
NP8 = N_PAD // 8
E_PAD = 3211264
R = E_PAD // 128
NC, NS = 2, 16
ROWS_PER_TILE = R // (NC * NS)
STRIPE = N_PAD // NS
SP8 = STRIPE // 8
GD = 8
MD = ROWS_PER_TILE // GD
BD = GD * 128
GP = 4
MP = ROWS_PER_TILE // GP
BP = GP * 128
CHUNK = 224
PB = 128
GRID = NP8 // PB

_MESH = plsc.VectorSubcoreMesh(
    core_axis_name="c", subcore_axis_name="s", num_cores=NC, num_subcores=NS)
_SC_PARAMS = pltpu.CompilerParams(use_tc_tiling_on_sc=False)


def _wid():
    return lax.axis_index("s") * NC + lax.axis_index("c")


def _zero_acc(zp_hbm, acc, r0):
    for j in range(STRIPE // CHUNK):
        pltpu.sync_copy(zp_hbm, acc.at[pl.ds(r0 + j * CHUNK, CHUNK)])


def _repack_writeback(acc, abuf, pbuf, out_hbm, c, s, r0):
    p0 = s * SP8

    def chunk(t, carry):
        pltpu.sync_copy(acc.at[pl.ds(r0 + t * CHUNK, CHUNK)], abuf)
        for r8 in range(CHUNK // 8):
            for u in range(8):
                pbuf[r8, pl.ds(16 * u, 16)] = abuf[8 * r8 + u, :]
        pltpu.sync_copy(pbuf, out_hbm.at[c, pl.ds(p0 + t * (CHUNK // 8),
                                                  CHUNK // 8)])
        return carry

    lax.fori_loop(0, STRIPE // CHUNK, chunk, 0)


@functools.partial(
    pl.kernel,
    out_type=jax.ShapeDtypeStruct((NC, NP8, 128), jnp.float32),
    mesh=_MESH,
    compiler_params=_SC_PARAMS,
    scratch_types=[
        pltpu.VMEM_SHARED((N_PAD, D), jnp.float32),
        pltpu.VMEM((BD, D), jnp.float32),
        pltpu.VMEM((2, BD), jnp.int32),
        pltpu.VMEM((CHUNK, D), jnp.float32),
        pltpu.VMEM((CHUNK // 8, 128), jnp.float32),
        pltpu.SemaphoreType.DMA,
        pltpu.SemaphoreType.DMA,
    ],
)
def _deg_kernel(dst_hbm, zp_hbm, ones_hbm, out_hbm, acc, ones_v, dbuf,
                abuf, pbuf, isem, ssem):
    c = lax.axis_index("c")
    s = lax.axis_index("s")
    pltpu.sync_copy(ones_hbm, ones_v)
    r0 = s * STRIPE
    _zero_acc(zp_hbm, acc, r0)
    plsc.subcore_barrier()
    rows0 = _wid() * ROWS_PER_TILE
    last = rows0 + (MD - 1) * GD

    def idx_fire(base, b):
        pltpu.async_copy(dst_hbm.at[pl.ds(base * 128, BD)], dbuf.at[b], isem)

    def idx_wait(b):
        pltpu.make_async_copy(
            dst_hbm.at[pl.ds(0, BD)], dbuf.at[b], isem).wait()

    idx_fire(rows0, 0)
    idx_fire(rows0 + GD, 1)

    def outer(k, carry):
        del k
        for u in range(2):
            b = u
            idx_wait(b)
            pltpu.async_copy(ones_v, acc.at[dbuf.at[b]], ssem, add=True)
            pltpu.make_async_copy(ones_v, acc.at[dbuf.at[b]], ssem).wait()
            g = carry + u
            nbase = lax.min(rows0 + (g + 2) * GD, last)
            idx_fire(nbase, b)
        return carry + 2

    lax.fori_loop(0, MD // 2, outer, 0)
    idx_wait(0)
    idx_wait(1)
    plsc.subcore_barrier()
    _repack_writeback(acc, abuf, pbuf, out_hbm, c, s, r0)


@functools.partial(
    pl.kernel,
    out_type=jax.ShapeDtypeStruct((NC, NP8, 128), jnp.float32),
    mesh=_MESH,
    compiler_params=_SC_PARAMS,
    scratch_types=[
        pltpu.VMEM_SHARED((N_PAD, D), jnp.float32),
        pltpu.VMEM((2, BP), jnp.int32),
        pltpu.VMEM((2, BP), jnp.int32),
        pltpu.VMEM((2, BP, D), jnp.float32),
        pltpu.VMEM((CHUNK, D), jnp.float32),
        pltpu.VMEM((CHUNK // 8, 128), jnp.float32),
        pltpu.SemaphoreType.DMA,
        pltpu.SemaphoreType.DMA,
        pltpu.SemaphoreType.DMA,
        pltpu.SemaphoreType.DMA,
        pltpu.SemaphoreType.DMA,
    ],
)
def _prop_kernel(table_hbm, src_hbm, dst_hbm, zp_hbm, out_hbm,
                 acc, sbuf, dbuf, rows, abuf, pbuf,
                 isem, gsem0, gsem1, ssem0, ssem1):
    c = lax.axis_index("c")
    s = lax.axis_index("s")
    gsem = (gsem0, gsem1)
    ssem = (ssem0, ssem1)
    r0 = s * STRIPE
    _zero_acc(zp_hbm, acc, r0)
    plsc.subcore_barrier()
    rows0 = _wid() * ROWS_PER_TILE
    last = rows0 + (MP - 1) * GP

    def idx_fire(base, b):
        pltpu.async_copy(src_hbm.at[pl.ds(base * 128, BP)], sbuf.at[b], isem)
        pltpu.async_copy(dst_hbm.at[pl.ds(base * 128, BP)], dbuf.at[b], isem)

    def idx_wait(b):
        pltpu.make_async_copy(src_hbm.at[pl.ds(0, BP)], sbuf.at[b],
                              isem).wait()
        pltpu.make_async_copy(dst_hbm.at[pl.ds(0, BP)], dbuf.at[b],
                              isem).wait()

    def gather_fire(b):
        pltpu.async_copy(table_hbm.at[sbuf.at[b]], rows.at[b], gsem[b])

    def gather_drain(b):
        pltpu.make_async_copy(table_hbm.at[sbuf.at[b]], rows.at[b],
                              gsem[b]).wait()

    def scatter_fire(b):
        pltpu.async_copy(rows.at[b], acc.at[dbuf.at[b]], ssem[b], add=True)

    def scatter_drain(b):
        pltpu.make_async_copy(rows.at[b], acc.at[dbuf.at[b]], ssem[b]).wait()

    idx_fire(rows0, 0)
    idx_wait(0)
    gather_fire(0)
    idx_fire(rows0 + GP, 1)

    def outer(k, carry):
        del k
        for u in range(2):
            b = u
            bn = 1 - u
            g = carry + u
            idx_wait(bn)
            gather_drain(b)
            gather_fire(bn)
            scatter_fire(b)
            scatter_drain(b)
            nbase = lax.min(rows0 + (g + 2) * GP, last)
            idx_fire(nbase, b)
        return carry + 2

    lax.fori_loop(0, MP // 2, outer, 0)
    idx_wait(0)
    gather_drain(0)
    plsc.subcore_barrier()
    _repack_writeback(acc, abuf, pbuf, out_hbm, c, s, r0)


def _row_spec(width=128):
    return pl.BlockSpec((PB, width), lambda i: (i, 0))


def _part_spec():
    return pl.BlockSpec((NC, PB, 128), lambda i: (0, i, 0))


def _full_spec(a, b):
    return pl.BlockSpec((a, b), lambda i: (0, 0))


def _tc1_body(deg_ref, x_ref, w1_ref, dis_ref, h1s_ref):
    d = 1.0 + deg_ref[0] + deg_ref[1]
    dis = lax.rsqrt(d)
    dis_ref[...] = dis
    h1s_ref[...] = dis * jnp.dot(x_ref[...], w1_ref[...],
                                 preferred_element_type=jnp.float32)


def _tc_mid_body(acc_ref, hs_ref, dis_ref, w_ref, b_ref, out_ref, *,
                 final_scale):
    dis = dis_ref[...]
    tot = acc_ref[0] + acc_ref[1] + hs_ref[...]
    o = jax.nn.relu(dis * tot + b_ref[...])
    if final_scale:
        out_ref[...] = dis * o
    else:
        out_ref[...] = dis * jnp.dot(o, w_ref[...],
                                     preferred_element_type=jnp.float32)


def _tc4_body(acc_ref, t3s_ref, dis_ref, w3_ref, b3_ref, out_ref):
    dis = dis_ref[...]
    p = dis * (acc_ref[0] + acc_ref[1] + t3s_ref[...])
    z = jnp.dot(p, w3_ref[...], preferred_element_type=jnp.float32)
    out_ref[...] = jax.nn.sigmoid(z + b3_ref[...])


def kernel(x, edge_index, W1, b1, W2, b2, W3, b3):
    f32 = jnp.float32
    ei = edge_index.astype(jnp.int32)
    pad = E_PAD - E
    padv = jnp.full((2, pad), N, jnp.int32)
    eip = jnp.concatenate([ei, padv], axis=1)
    src1 = eip[0]
    dst1 = eip[1]
    x_p = jnp.pad(x.astype(f32), ((0, N_PAD - N), (0, 0))).reshape(NP8, 128)

    zp = jnp.zeros((CHUNK, D), f32)
    ones16 = jnp.ones((BD, D), f32)

    eye8 = jnp.eye(8, dtype=f32)
    w1b = jnp.kron(eye8, W1.astype(f32))
    w2b = jnp.kron(eye8, W2.astype(f32))
    w3b = jnp.kron(eye8, W3.astype(f32))
    b1t = jnp.tile(b1.astype(f32), 8).reshape(1, 128)
    b2t = jnp.tile(b2.astype(f32), 8).reshape(1, 128)
    b3t = jnp.tile(b3.astype(f32), 8).reshape(1, 8)

    deg_parts = _deg_kernel(dst1, zp, ones16)

    dis, h1s = pl.pallas_call(
        _tc1_body,
        grid=(GRID,),
        in_specs=[_part_spec(), _row_spec(), _full_spec(128, 128)],
        out_specs=[_row_spec(), _row_spec()],
        out_shape=[jax.ShapeDtypeStruct((NP8, 128), f32),
                   jax.ShapeDtypeStruct((NP8, 128), f32)],
    )(deg_parts, x_p, w1b)

    acc1 = _prop_kernel(h1s.reshape(N_PAD, D), src1, dst1, zp)

    h2s = pl.pallas_call(
        functools.partial(_tc_mid_body, final_scale=False),
        grid=(GRID,),
        in_specs=[_part_spec(), _row_spec(), _row_spec(),
                  _full_spec(128, 128), _full_spec(1, 128)],
        out_specs=_row_spec(),
        out_shape=jax.ShapeDtypeStruct((NP8, 128), f32),
    )(acc1, h1s, dis, w2b, b1t)

    acc2 = _prop_kernel(h2s.reshape(N_PAD, D), src1, dst1, zp)

    t3s = pl.pallas_call(
        functools.partial(_tc_mid_body, final_scale=True),
        grid=(GRID,),
        in_specs=[_part_spec(), _row_spec(), _row_spec(),
                  _full_spec(128, 128), _full_spec(1, 128)],
        out_specs=_row_spec(),
        out_shape=jax.ShapeDtypeStruct((NP8, 128), f32),
    )(acc2, h2s, dis, w2b, b2t)

    acc3 = _prop_kernel(t3s.reshape(N_PAD, D), src1, dst1, zp)

    out8 = pl.pallas_call(
        _tc4_body,
        grid=(GRID,),
        in_specs=[_part_spec(), _row_spec(), _row_spec(),
                  _full_spec(128, 8), _full_spec(1, 8)],
        out_specs=_row_spec(8),
        out_shape=jax.ShapeDtypeStruct((NP8, 8), f32),
    )(acc3, t3s, dis, w3b, b3t)

    return out8.reshape(N_PAD, 1)[:N]


# speedup vs baseline: 80.2472x; 1.1023x over previous
import functools

import jax
import jax.numpy as jnp
from jax import lax
from jax.experimental import pallas as pl
from jax.experimental.pallas import tpu as pltpu
from jax.experimental.pallas import tpu_sc as plsc

N = 100000
E = 3200000
D = 16

N_PAD = 100352
NP8 = N_PAD // 8
E_PAD = 3211264
R = E_PAD // 128
NC, NS = 2, 16
ROWS_PER_TILE = R // (NC * NS)
STRIPE = N_PAD // NS
SP8 = STRIPE // 8
GD = 8
MD = ROWS_PER_TILE // GD
BD = GD * 128
GP = 4
MP = ROWS_PER_TILE // GP
BP = GP * 128
CHUNK = 224
PB = 448
GRID = NP8 // PB

_MESH = plsc.VectorSubcoreMesh(
    core_axis_name="c", subcore_axis_name="s", num_cores=NC, num_subcores=NS)
_SC_PARAMS = pltpu.CompilerParams(use_tc_tiling_on_sc=False)


def _wid():
    return lax.axis_index("s") * NC + lax.axis_index("c")


def _zero_acc(zp_hbm, acc, r0):
    for j in range(STRIPE // CHUNK):
        pltpu.sync_copy(zp_hbm, acc.at[pl.ds(r0 + j * CHUNK, CHUNK)])


def _repack_writeback(acc, abuf, pbuf, out_hbm, c, s, r0):
    p0 = s * SP8

    def chunk(t, carry):
        pltpu.sync_copy(acc.at[pl.ds(r0 + t * CHUNK, CHUNK)], abuf)
        for r8 in range(CHUNK // 8):
            for u in range(8):
                pbuf[r8, pl.ds(16 * u, 16)] = abuf[8 * r8 + u, :]
        pltpu.sync_copy(pbuf, out_hbm.at[c, pl.ds(p0 + t * (CHUNK // 8),
                                                  CHUNK // 8)])
        return carry

    lax.fori_loop(0, STRIPE // CHUNK, chunk, 0)


@functools.partial(
    pl.kernel,
    out_type=jax.ShapeDtypeStruct((NC, NP8, 128), jnp.float32),
    mesh=_MESH,
    compiler_params=_SC_PARAMS,
    scratch_types=[
        pltpu.VMEM_SHARED((N_PAD, D), jnp.float32),
        pltpu.VMEM((BD, D), jnp.float32),
        pltpu.VMEM((2, BD), jnp.int32),
        pltpu.VMEM((CHUNK, D), jnp.float32),
        pltpu.VMEM((CHUNK // 8, 128), jnp.float32),
        pltpu.SemaphoreType.DMA,
        pltpu.SemaphoreType.DMA,
    ],
)
def _deg_kernel(dst_hbm, zp_hbm, ones_hbm, out_hbm, acc, ones_v, dbuf,
                abuf, pbuf, isem, ssem):
    c = lax.axis_index("c")
    s = lax.axis_index("s")
    pltpu.sync_copy(ones_hbm, ones_v)
    r0 = s * STRIPE
    _zero_acc(zp_hbm, acc, r0)
    plsc.subcore_barrier()
    rows0 = _wid() * ROWS_PER_TILE
    last = rows0 + (MD - 1) * GD

    def idx_fire(base, b):
        pltpu.async_copy(dst_hbm.at[pl.ds(base * 128, BD)], dbuf.at[b], isem)

    def idx_wait(b):
        pltpu.make_async_copy(
            dst_hbm.at[pl.ds(0, BD)], dbuf.at[b], isem).wait()

    idx_fire(rows0, 0)
    idx_fire(rows0 + GD, 1)

    def outer(k, carry):
        del k
        for u in range(2):
            b = u
            idx_wait(b)
            pltpu.async_copy(ones_v, acc.at[dbuf.at[b]], ssem, add=True)
            pltpu.make_async_copy(ones_v, acc.at[dbuf.at[b]], ssem).wait()
            g = carry + u
            nbase = lax.min(rows0 + (g + 2) * GD, last)
            idx_fire(nbase, b)
        return carry + 2

    lax.fori_loop(0, MD // 2, outer, 0)
    idx_wait(0)
    idx_wait(1)
    plsc.subcore_barrier()
    _repack_writeback(acc, abuf, pbuf, out_hbm, c, s, r0)


@functools.partial(
    pl.kernel,
    out_type=jax.ShapeDtypeStruct((NC, NP8, 128), jnp.float32),
    mesh=_MESH,
    compiler_params=_SC_PARAMS,
    scratch_types=[
        pltpu.VMEM_SHARED((N_PAD, D), jnp.float32),
        pltpu.VMEM((2, BP), jnp.int32),
        pltpu.VMEM((2, BP), jnp.int32),
        pltpu.VMEM((2, BP, D), jnp.float32),
        pltpu.VMEM((CHUNK, D), jnp.float32),
        pltpu.VMEM((CHUNK // 8, 128), jnp.float32),
        pltpu.SemaphoreType.DMA,
        pltpu.SemaphoreType.DMA,
        pltpu.SemaphoreType.DMA,
        pltpu.SemaphoreType.DMA,
        pltpu.SemaphoreType.DMA,
    ],
)
def _prop_kernel(table_hbm, src_hbm, dst_hbm, zp_hbm, out_hbm,
                 acc, sbuf, dbuf, rows, abuf, pbuf,
                 isem, gsem0, gsem1, ssem0, ssem1):
    c = lax.axis_index("c")
    s = lax.axis_index("s")
    gsem = (gsem0, gsem1)
    ssem = (ssem0, ssem1)
    r0 = s * STRIPE
    _zero_acc(zp_hbm, acc, r0)
    plsc.subcore_barrier()
    rows0 = _wid() * ROWS_PER_TILE
    last = rows0 + (MP - 1) * GP

    def idx_fire(base, b):
        pltpu.async_copy(src_hbm.at[pl.ds(base * 128, BP)], sbuf.at[b], isem)
        pltpu.async_copy(dst_hbm.at[pl.ds(base * 128, BP)], dbuf.at[b], isem)

    def idx_wait(b):
        pltpu.make_async_copy(src_hbm.at[pl.ds(0, BP)], sbuf.at[b],
                              isem).wait()
        pltpu.make_async_copy(dst_hbm.at[pl.ds(0, BP)], dbuf.at[b],
                              isem).wait()

    def gather_fire(b):
        pltpu.async_copy(table_hbm.at[sbuf.at[b]], rows.at[b], gsem[b])

    def gather_drain(b):
        pltpu.make_async_copy(table_hbm.at[sbuf.at[b]], rows.at[b],
                              gsem[b]).wait()

    def scatter_fire(b):
        pltpu.async_copy(rows.at[b], acc.at[dbuf.at[b]], ssem[b], add=True)

    def scatter_drain(b):
        pltpu.make_async_copy(rows.at[b], acc.at[dbuf.at[b]], ssem[b]).wait()

    idx_fire(rows0, 0)
    idx_wait(0)
    gather_fire(0)
    idx_fire(rows0 + GP, 1)

    def outer(k, carry):
        del k
        for u in range(2):
            b = u
            bn = 1 - u
            g = carry + u
            idx_wait(bn)
            gather_drain(b)
            scatter_fire(b)
            gather_fire(bn)
            scatter_drain(b)
            nbase = lax.min(rows0 + (g + 2) * GP, last)
            idx_fire(nbase, b)
        return carry + 2

    lax.fori_loop(0, MP // 2, outer, 0)
    idx_wait(0)
    gather_drain(0)
    plsc.subcore_barrier()
    _repack_writeback(acc, abuf, pbuf, out_hbm, c, s, r0)


def _row_spec(width=128):
    return pl.BlockSpec((PB, width), lambda i: (i, 0))


def _part_spec():
    return pl.BlockSpec((NC, PB, 128), lambda i: (0, i, 0))


def _full_spec(a, b):
    return pl.BlockSpec((a, b), lambda i: (0, 0))


def _tc1_body(deg_ref, x_ref, w1_ref, dis_ref, h1s_ref):
    d = 1.0 + deg_ref[0] + deg_ref[1]
    dis = lax.rsqrt(d)
    dis_ref[...] = dis
    h1s_ref[...] = dis * jnp.dot(x_ref[...], w1_ref[...],
                                 preferred_element_type=jnp.float32)


def _tc_mid_body(acc_ref, hs_ref, dis_ref, w_ref, b_ref, out_ref, *,
                 final_scale):
    dis = dis_ref[...]
    tot = acc_ref[0] + acc_ref[1] + hs_ref[...]
    o = jax.nn.relu(dis * tot + b_ref[...])
    if final_scale:
        out_ref[...] = dis * o
    else:
        out_ref[...] = dis * jnp.dot(o, w_ref[...],
                                     preferred_element_type=jnp.float32)


def _tc4_body(acc_ref, t3s_ref, dis_ref, w3_ref, b3_ref, out_ref):
    dis = dis_ref[...]
    p = dis * (acc_ref[0] + acc_ref[1] + t3s_ref[...])
    z = jnp.dot(p, w3_ref[...], preferred_element_type=jnp.float32)
    out_ref[...] = jax.nn.sigmoid(z + b3_ref[...])


def kernel(x, edge_index, W1, b1, W2, b2, W3, b3):
    f32 = jnp.float32
    ei = edge_index.astype(jnp.int32)
    pad = E_PAD - E
    padv = jnp.full((2, pad), N, jnp.int32)
    eip = jnp.concatenate([ei, padv], axis=1)
    src1 = eip[0]
    dst1 = eip[1]
    x_p = jnp.pad(x.astype(f32), ((0, N_PAD - N), (0, 0))).reshape(NP8, 128)

    zp = jnp.zeros((CHUNK, D), f32)
    ones16 = jnp.ones((BD, D), f32)

    eye8 = jnp.eye(8, dtype=f32)
    w1b = jnp.kron(eye8, W1.astype(f32))
    w2b = jnp.kron(eye8, W2.astype(f32))
    w3b = jnp.kron(eye8, W3.astype(f32))
    b1t = jnp.tile(b1.astype(f32), 8).reshape(1, 128)
    b2t = jnp.tile(b2.astype(f32), 8).reshape(1, 128)
    b3t = jnp.tile(b3.astype(f32), 8).reshape(1, 8)

    deg_parts = _deg_kernel(dst1, zp, ones16)

    dis, h1s = pl.pallas_call(
        _tc1_body,
        grid=(GRID,),
        in_specs=[_part_spec(), _row_spec(), _full_spec(128, 128)],
        out_specs=[_row_spec(), _row_spec()],
        out_shape=[jax.ShapeDtypeStruct((NP8, 128), f32),
                   jax.ShapeDtypeStruct((NP8, 128), f32)],
    )(deg_parts, x_p, w1b)

    acc1 = _prop_kernel(h1s.reshape(N_PAD, D), src1, dst1, zp)

    h2s = pl.pallas_call(
        functools.partial(_tc_mid_body, final_scale=False),
        grid=(GRID,),
        in_specs=[_part_spec(), _row_spec(), _row_spec(),
                  _full_spec(128, 128), _full_spec(1, 128)],
        out_specs=_row_spec(),
        out_shape=jax.ShapeDtypeStruct((NP8, 128), f32),
    )(acc1, h1s, dis, w2b, b1t)

    acc2 = _prop_kernel(h2s.reshape(N_PAD, D), src1, dst1, zp)

    t3s = pl.pallas_call(
        functools.partial(_tc_mid_body, final_scale=True),
        grid=(GRID,),
        in_specs=[_part_spec(), _row_spec(), _row_spec(),
                  _full_spec(128, 128), _full_spec(1, 128)],
        out_specs=_row_spec(),
        out_shape=jax.ShapeDtypeStruct((NP8, 128), f32),
    )(acc2, h2s, dis, w2b, b2t)

    acc3 = _prop_kernel(t3s.reshape(N_PAD, D), src1, dst1, zp)

    out8 = pl.pallas_call(
        _tc4_body,
        grid=(GRID,),
        in_specs=[_part_spec(), _row_spec(), _row_spec(),
                  _full_spec(128, 8), _full_spec(1, 8)],
        out_specs=_row_spec(8),
        out_shape=jax.ShapeDtypeStruct((NP8, 8), f32),
    )(acc3, t3s, dis, w3b, b3t)

    return out8.reshape(N_PAD, 1)[:N]


# speedup vs baseline: 85.9384x; 1.0709x over previous
import functools

import jax
import jax.numpy as jnp
from jax import lax
from jax.experimental import pallas as pl
from jax.experimental.pallas import tpu as pltpu
from jax.experimental.pallas import tpu_sc as plsc

N = 100000
E = 3200000
D = 16

N_PAD = 100352
NP8 = N_PAD // 8
E_PAD = 3211264
R = E_PAD // 128
NC, NS = 2, 16
ROWS_PER_TILE = R // (NC * NS)
STRIPE = N_PAD // NS
SP8 = STRIPE // 8
GD = 8
MD = ROWS_PER_TILE // GD
BD = GD * 128
GP = 4
MP = ROWS_PER_TILE // GP
BP = GP * 128
CHUNK = 224
PB = 448
GRID = NP8 // PB

_MESH = plsc.VectorSubcoreMesh(
    core_axis_name="c", subcore_axis_name="s", num_cores=NC, num_subcores=NS)
_SC_PARAMS = pltpu.CompilerParams(use_tc_tiling_on_sc=False)


def _wid():
    return lax.axis_index("s") * NC + lax.axis_index("c")


def _zero_acc(zp_hbm, acc, r0):
    for j in range(STRIPE // CHUNK):
        pltpu.sync_copy(zp_hbm, acc.at[pl.ds(r0 + j * CHUNK, CHUNK)])


def _repack_writeback(acc, abuf, pbuf, out_hbm, c, s, r0):
    p0 = s * SP8

    def chunk(t, carry):
        pltpu.sync_copy(acc.at[pl.ds(r0 + t * CHUNK, CHUNK)], abuf)
        for r8 in range(CHUNK // 8):
            for u in range(8):
                pbuf[r8, pl.ds(16 * u, 16)] = abuf[8 * r8 + u, :]
        pltpu.sync_copy(pbuf, out_hbm.at[c, pl.ds(p0 + t * (CHUNK // 8),
                                                  CHUNK // 8)])
        return carry

    lax.fori_loop(0, STRIPE // CHUNK, chunk, 0)


@functools.partial(
    pl.kernel,
    out_type=jax.ShapeDtypeStruct((NC, NP8, 128), jnp.float32),
    mesh=_MESH,
    compiler_params=_SC_PARAMS,
    scratch_types=[
        pltpu.VMEM_SHARED((N_PAD, D), jnp.float32),
        pltpu.VMEM((BD, D), jnp.float32),
        pltpu.VMEM((2, BD), jnp.int32),
        pltpu.VMEM((CHUNK, D), jnp.float32),
        pltpu.VMEM((CHUNK // 8, 128), jnp.float32),
        pltpu.SemaphoreType.DMA,
        pltpu.SemaphoreType.DMA,
    ],
)
def _deg_kernel(dst_hbm, zp_hbm, ones_hbm, out_hbm, acc, ones_v, dbuf,
                abuf, pbuf, isem, ssem):
    c = lax.axis_index("c")
    s = lax.axis_index("s")
    pltpu.sync_copy(ones_hbm, ones_v)
    r0 = s * STRIPE
    _zero_acc(zp_hbm, acc, r0)
    plsc.subcore_barrier()
    rows0 = _wid() * ROWS_PER_TILE
    last = rows0 + (MD - 1) * GD

    def idx_fire(base, b):
        pltpu.async_copy(dst_hbm.at[pl.ds(base * 128, BD)], dbuf.at[b], isem)

    def idx_wait(b):
        pltpu.make_async_copy(
            dst_hbm.at[pl.ds(0, BD)], dbuf.at[b], isem).wait()

    idx_fire(rows0, 0)
    idx_fire(rows0 + GD, 1)

    def outer(k, carry):
        del k
        for u in range(2):
            b = u
            idx_wait(b)
            pltpu.async_copy(ones_v, acc.at[dbuf.at[b]], ssem, add=True)
            pltpu.make_async_copy(ones_v, acc.at[dbuf.at[b]], ssem).wait()
            g = carry + u
            nbase = lax.min(rows0 + (g + 2) * GD, last)
            idx_fire(nbase, b)
        return carry + 2

    lax.fori_loop(0, MD // 2, outer, 0)
    idx_wait(0)
    idx_wait(1)
    plsc.subcore_barrier()
    _repack_writeback(acc, abuf, pbuf, out_hbm, c, s, r0)


@functools.partial(
    pl.kernel,
    out_type=jax.ShapeDtypeStruct((NC, NP8, 128), jnp.float32),
    mesh=_MESH,
    compiler_params=_SC_PARAMS,
    scratch_types=[
        pltpu.VMEM_SHARED((N_PAD, D), jnp.float32),
        pltpu.VMEM((2, BP), jnp.int32),
        pltpu.VMEM((2, BP), jnp.int32),
        pltpu.VMEM((2, BP, D), jnp.float32),
        pltpu.VMEM((CHUNK, D), jnp.float32),
        pltpu.VMEM((CHUNK // 8, 128), jnp.float32),
        pltpu.SemaphoreType.DMA,
        pltpu.SemaphoreType.DMA,
        pltpu.SemaphoreType.DMA,
        pltpu.SemaphoreType.DMA,
        pltpu.SemaphoreType.DMA,
    ],
)
def _prop_kernel(table_hbm, src_hbm, dst_hbm, zp_hbm, out_hbm,
                 acc, sbuf, dbuf, rows, abuf, pbuf,
                 isem, gsem0, gsem1, ssem0, ssem1):
    c = lax.axis_index("c")
    s = lax.axis_index("s")
    gsem = (gsem0, gsem1)
    ssem = (ssem0, ssem1)
    r0 = s * STRIPE
    _zero_acc(zp_hbm, acc, r0)
    plsc.subcore_barrier()
    rows0 = _wid() * ROWS_PER_TILE
    last = rows0 + (MP - 1) * GP

    def idx_fire(base, b):
        pltpu.async_copy(src_hbm.at[pl.ds(base * 128, BP)], sbuf.at[b], isem)
        pltpu.async_copy(dst_hbm.at[pl.ds(base * 128, BP)], dbuf.at[b], isem)

    def idx_wait(b):
        pltpu.make_async_copy(src_hbm.at[pl.ds(0, BP)], sbuf.at[b],
                              isem).wait()
        pltpu.make_async_copy(dst_hbm.at[pl.ds(0, BP)], dbuf.at[b],
                              isem).wait()

    def gather_fire(b):
        pltpu.async_copy(table_hbm.at[sbuf.at[b]], rows.at[b], gsem[b])

    def gather_drain(b):
        pltpu.make_async_copy(table_hbm.at[sbuf.at[b]], rows.at[b],
                              gsem[b]).wait()

    def scatter_fire(b):
        pltpu.async_copy(rows.at[b], acc.at[dbuf.at[b]], ssem[b], add=True)

    def scatter_drain(b):
        pltpu.make_async_copy(rows.at[b], acc.at[dbuf.at[b]], ssem[b]).wait()

    idx_fire(rows0, 0)
    idx_wait(0)
    gather_fire(0)
    idx_fire(rows0 + GP, 1)

    def outer(k, carry):
        del k
        for u in range(2):
            b = u
            bn = 1 - u
            g = carry + u
            idx_wait(bn)
            gather_fire(bn)
            gather_drain(b)
            scatter_fire(b)
            scatter_drain(b)
            nbase = lax.min(rows0 + (g + 2) * GP, last)
            idx_fire(nbase, b)
        return carry + 2

    lax.fori_loop(0, MP // 2, outer, 0)
    idx_wait(0)
    gather_drain(0)
    plsc.subcore_barrier()
    _repack_writeback(acc, abuf, pbuf, out_hbm, c, s, r0)


def _row_spec(width=128):
    return pl.BlockSpec((PB, width), lambda i: (i, 0))


def _part_spec():
    return pl.BlockSpec((NC, PB, 128), lambda i: (0, i, 0))


def _full_spec(a, b):
    return pl.BlockSpec((a, b), lambda i: (0, 0))


def _tc1_body(deg_ref, x_ref, w1_ref, dis_ref, h1s_ref):
    d = 1.0 + deg_ref[0] + deg_ref[1]
    dis = lax.rsqrt(d)
    dis_ref[...] = dis
    h1s_ref[...] = dis * jnp.dot(x_ref[...], w1_ref[...],
                                 preferred_element_type=jnp.float32)


def _tc_mid_body(acc_ref, hs_ref, dis_ref, w_ref, b_ref, out_ref, *,
                 final_scale):
    dis = dis_ref[...]
    tot = acc_ref[0] + acc_ref[1] + hs_ref[...]
    o = jax.nn.relu(dis * tot + b_ref[...])
    if final_scale:
        out_ref[...] = dis * o
    else:
        out_ref[...] = dis * jnp.dot(o, w_ref[...],
                                     preferred_element_type=jnp.float32)


def _tc4_body(acc_ref, t3s_ref, dis_ref, w3_ref, b3_ref, out_ref):
    dis = dis_ref[...]
    p = dis * (acc_ref[0] + acc_ref[1] + t3s_ref[...])
    z = jnp.dot(p, w3_ref[...], preferred_element_type=jnp.float32)
    out_ref[...] = jax.nn.sigmoid(z + b3_ref[...])


def kernel(x, edge_index, W1, b1, W2, b2, W3, b3):
    f32 = jnp.float32
    ei = edge_index.astype(jnp.int32)
    pad = E_PAD - E
    padv = jnp.full((2, pad), N, jnp.int32)
    eip = jnp.concatenate([ei, padv], axis=1)
    src1 = eip[0]
    dst1 = eip[1]
    x_p = jnp.pad(x.astype(f32), ((0, N_PAD - N), (0, 0))).reshape(NP8, 128)

    zp = jnp.zeros((CHUNK, D), f32)
    ones16 = jnp.ones((BD, D), f32)

    eye8 = jnp.eye(8, dtype=f32)
    w1b = jnp.kron(eye8, W1.astype(f32))
    w2b = jnp.kron(eye8, W2.astype(f32))
    w3b = jnp.kron(eye8, W3.astype(f32))
    b1t = jnp.tile(b1.astype(f32), 8).reshape(1, 128)
    b2t = jnp.tile(b2.astype(f32), 8).reshape(1, 128)
    b3t = jnp.tile(b3.astype(f32), 8).reshape(1, 8)

    deg_parts = _deg_kernel(dst1, zp, ones16)

    dis, h1s = pl.pallas_call(
        _tc1_body,
        grid=(GRID,),
        in_specs=[_part_spec(), _row_spec(), _full_spec(128, 128)],
        out_specs=[_row_spec(), _row_spec()],
        out_shape=[jax.ShapeDtypeStruct((NP8, 128), f32),
                   jax.ShapeDtypeStruct((NP8, 128), f32)],
    )(deg_parts, x_p, w1b)

    acc1 = _prop_kernel(h1s.reshape(N_PAD, D), src1, dst1, zp)

    h2s = pl.pallas_call(
        functools.partial(_tc_mid_body, final_scale=False),
        grid=(GRID,),
        in_specs=[_part_spec(), _row_spec(), _row_spec(),
                  _full_spec(128, 128), _full_spec(1, 128)],
        out_specs=_row_spec(),
        out_shape=jax.ShapeDtypeStruct((NP8, 128), f32),
    )(acc1, h1s, dis, w2b, b1t)

    acc2 = _prop_kernel(h2s.reshape(N_PAD, D), src1, dst1, zp)

    t3s = pl.pallas_call(
        functools.partial(_tc_mid_body, final_scale=True),
        grid=(GRID,),
        in_specs=[_part_spec(), _row_spec(), _row_spec(),
                  _full_spec(128, 128), _full_spec(1, 128)],
        out_specs=_row_spec(),
        out_shape=jax.ShapeDtypeStruct((NP8, 128), f32),
    )(acc2, h2s, dis, w2b, b2t)

    acc3 = _prop_kernel(t3s.reshape(N_PAD, D), src1, dst1, zp)

    out8 = pl.pallas_call(
        _tc4_body,
        grid=(GRID,),
        in_specs=[_part_spec(), _row_spec(), _row_spec(),
                  _full_spec(128, 8), _full_spec(1, 8)],
        out_specs=_row_spec(8),
        out_shape=jax.ShapeDtypeStruct((NP8, 8), f32),
    )(acc3, t3s, dis, w3b, b3t)

    return out8.reshape(N_PAD, 1)[:N]


# speedup vs baseline: 87.5718x; 1.0190x over previous
import functools

import jax
import jax.numpy as jnp
from jax import lax
from jax.experimental import pallas as pl
from jax.experimental.pallas import tpu as pltpu
from jax.experimental.pallas import tpu_sc as plsc

N = 100000
E = 3200000
D = 16

N_PAD = 100352
NP8 = N_PAD // 8
E_PAD = 3211264
R = E_PAD // 128
NC, NS = 2, 16
ROWS_PER_TILE = R // (NC * NS)
STRIPE = N_PAD // NS
SP8 = STRIPE // 8
GD = 8
MD = ROWS_PER_TILE // GD
BD = GD * 128
GP = 4
MP = ROWS_PER_TILE // GP
BP = GP * 128
CHUNK = 224
PB = 448
GRID = NP8 // PB

_MESH = plsc.VectorSubcoreMesh(
    core_axis_name="c", subcore_axis_name="s", num_cores=NC, num_subcores=NS)
_SC_PARAMS = pltpu.CompilerParams(use_tc_tiling_on_sc=False)


def _wid():
    return lax.axis_index("s") * NC + lax.axis_index("c")


def _zero_acc(zp_hbm, acc, r0):
    for j in range(STRIPE // CHUNK):
        pltpu.sync_copy(zp_hbm, acc.at[pl.ds(r0 + j * CHUNK, CHUNK)])


def _repack_writeback(acc, abuf, pbuf, out_hbm, c, s, r0):
    p0 = s * SP8

    def chunk(t, carry):
        pltpu.sync_copy(acc.at[pl.ds(r0 + t * CHUNK, CHUNK)], abuf)
        for r8 in range(CHUNK // 8):
            for u in range(8):
                pbuf[r8, pl.ds(16 * u, 16)] = abuf[8 * r8 + u, :]
        pltpu.sync_copy(pbuf, out_hbm.at[c, pl.ds(p0 + t * (CHUNK // 8),
                                                  CHUNK // 8)])
        return carry

    lax.fori_loop(0, STRIPE // CHUNK, chunk, 0)


@functools.partial(
    pl.kernel,
    out_type=jax.ShapeDtypeStruct((NC, NP8, 128), jnp.float32),
    mesh=_MESH,
    compiler_params=_SC_PARAMS,
    scratch_types=[
        pltpu.VMEM_SHARED((N_PAD, D), jnp.float32),
        pltpu.VMEM((BD, D), jnp.float32),
        pltpu.VMEM((2, BD), jnp.int32),
        pltpu.VMEM((CHUNK, D), jnp.float32),
        pltpu.VMEM((CHUNK // 8, 128), jnp.float32),
        pltpu.SemaphoreType.DMA,
        pltpu.SemaphoreType.DMA,
    ],
)
def _deg_kernel(dst_hbm, zp_hbm, ones_hbm, out_hbm, acc, ones_v, dbuf,
                abuf, pbuf, isem, ssem):
    c = lax.axis_index("c")
    s = lax.axis_index("s")
    pltpu.sync_copy(ones_hbm, ones_v)
    r0 = s * STRIPE
    _zero_acc(zp_hbm, acc, r0)
    plsc.subcore_barrier()
    rows0 = _wid() * ROWS_PER_TILE
    last = rows0 + (MD - 1) * GD

    def idx_fire(base, b):
        pltpu.async_copy(dst_hbm.at[pl.ds(base * 128, BD)], dbuf.at[b], isem)

    def idx_wait(b):
        pltpu.make_async_copy(
            dst_hbm.at[pl.ds(0, BD)], dbuf.at[b], isem).wait()

    idx_fire(rows0, 0)
    idx_fire(rows0 + GD, 1)

    def outer(k, carry):
        del k
        for u in range(2):
            b = u
            idx_wait(b)
            pltpu.async_copy(ones_v, acc.at[dbuf.at[b]], ssem, add=True)
            pltpu.make_async_copy(ones_v, acc.at[dbuf.at[b]], ssem).wait()
            g = carry + u
            nbase = lax.min(rows0 + (g + 2) * GD, last)
            idx_fire(nbase, b)
        return carry + 2

    lax.fori_loop(0, MD // 2, outer, 0)
    idx_wait(0)
    idx_wait(1)
    plsc.subcore_barrier()
    _repack_writeback(acc, abuf, pbuf, out_hbm, c, s, r0)


@functools.partial(
    pl.kernel,
    out_type=jax.ShapeDtypeStruct((NC, NP8, 128), jnp.float32),
    mesh=_MESH,
    compiler_params=_SC_PARAMS,
    scratch_types=[
        pltpu.VMEM_SHARED((N_PAD, D), jnp.float32),
        pltpu.VMEM((2, BP), jnp.int32),
        pltpu.VMEM((2, BP), jnp.int32),
        pltpu.VMEM((2, BP, D), jnp.float32),
        pltpu.VMEM((CHUNK, D), jnp.float32),
        pltpu.VMEM((CHUNK // 8, 128), jnp.float32),
        pltpu.SemaphoreType.DMA,
        pltpu.SemaphoreType.DMA,
        pltpu.SemaphoreType.DMA,
        pltpu.SemaphoreType.DMA,
        pltpu.SemaphoreType.DMA,
    ],
)
def _prop_kernel(table_hbm, src_hbm, dst_hbm, zp_hbm, out_hbm,
                 acc, sbuf, dbuf, rows, abuf, pbuf,
                 isem, gsem0, gsem1, ssem0, ssem1):
    c = lax.axis_index("c")
    s = lax.axis_index("s")
    gsem = (gsem0, gsem1)
    ssem = (ssem0, ssem1)
    r0 = s * STRIPE
    _zero_acc(zp_hbm, acc, r0)
    plsc.subcore_barrier()
    rows0 = _wid() * ROWS_PER_TILE
    last = rows0 + (MP - 1) * GP

    def idx_fire(base, b):
        pltpu.async_copy(src_hbm.at[pl.ds(base * 128, BP)], sbuf.at[b], isem)
        pltpu.async_copy(dst_hbm.at[pl.ds(base * 128, BP)], dbuf.at[b], isem)

    def idx_wait(b):
        pltpu.make_async_copy(src_hbm.at[pl.ds(0, BP)], sbuf.at[b],
                              isem).wait()
        pltpu.make_async_copy(dst_hbm.at[pl.ds(0, BP)], dbuf.at[b],
                              isem).wait()

    def gather_fire(b):
        pltpu.async_copy(table_hbm.at[sbuf.at[b]], rows.at[b], gsem[b])

    def gather_drain(b):
        pltpu.make_async_copy(table_hbm.at[sbuf.at[b]], rows.at[b],
                              gsem[b]).wait()

    def scatter_fire(b):
        pltpu.async_copy(rows.at[b], acc.at[dbuf.at[b]], ssem[b], add=True)

    def scatter_drain(b):
        pltpu.make_async_copy(rows.at[b], acc.at[dbuf.at[b]], ssem[b]).wait()

    idx_fire(rows0, 0)
    idx_wait(0)
    gather_fire(0)
    idx_fire(rows0 + GP, 1)

    def outer(k, carry):
        del k
        for u in range(2):
            b = u
            bn = 1 - u
            g = carry + u
            idx_wait(bn)
            gather_fire(bn)
            gather_drain(b)
            scatter_fire(b)
            scatter_drain(b)
            nbase = lax.min(rows0 + (g + 2) * GP, last)
            idx_fire(nbase, b)
        return carry + 2

    lax.fori_loop(0, MP // 2, outer, 0)
    idx_wait(0)
    gather_drain(0)
    plsc.subcore_barrier()
    _repack_writeback(acc, abuf, pbuf, out_hbm, c, s, r0)


def _row_spec(width=128):
    return pl.BlockSpec((PB, width), lambda i: (i, 0))


def _part_spec():
    return pl.BlockSpec((NC, PB, 128), lambda i: (0, i, 0))


def _full_spec(a, b):
    return pl.BlockSpec((a, b), lambda i: (0, 0))


def _tc1_body(deg_ref, x_ref, w1_ref, dis_ref, h1s_ref):
    d = 1.0 + deg_ref[0] + deg_ref[1]
    dis = lax.rsqrt(d)
    dis_ref[...] = dis
    h1s_ref[...] = dis * jnp.dot(x_ref[...], w1_ref[...],
                                 preferred_element_type=jnp.float32)


def _tc_mid_body(acc_ref, hs_ref, dis_ref, w_ref, b_ref, out_ref, *,
                 final_scale):
    dis = dis_ref[...]
    tot = acc_ref[0] + acc_ref[1] + hs_ref[...]
    o = jax.nn.relu(dis * tot + b_ref[...])
    if final_scale:
        out_ref[...] = dis * o
    else:
        out_ref[...] = dis * jnp.dot(o, w_ref[...],
                                     preferred_element_type=jnp.float32)


def _tc4_body(acc_ref, t3s_ref, dis_ref, w3_ref, b3_ref, out_ref):
    dis = dis_ref[...]
    p = dis * (acc_ref[0] + acc_ref[1] + t3s_ref[...])
    z = jnp.dot(p, w3_ref[...], preferred_element_type=jnp.float32)
    out_ref[...] = jax.nn.sigmoid(z + b3_ref[...])


def kernel(x, edge_index, W1, b1, W2, b2, W3, b3):
    f32 = jnp.float32
    ei = edge_index.astype(jnp.int32)
    pad = E_PAD - E
    padv = jnp.full((2, pad), N, jnp.int32)
    eip = jnp.concatenate([ei, padv], axis=1)
    src1 = eip[0]
    dst1 = eip[1]
    x_p = jnp.pad(x.astype(f32).reshape(N * D // 128, 128),
                  ((0, NP8 - N * D // 128), (0, 0)))

    zp = jnp.zeros((CHUNK, D), f32)
    ones16 = jnp.ones((BD, D), f32)

    eye8 = jnp.eye(8, dtype=f32)
    w1b = jnp.kron(eye8, W1.astype(f32))
    w2b = jnp.kron(eye8, W2.astype(f32))
    w3b = jnp.kron(eye8, W3.astype(f32))
    b1t = jnp.tile(b1.astype(f32), 8).reshape(1, 128)
    b2t = jnp.tile(b2.astype(f32), 8).reshape(1, 128)
    b3t = jnp.tile(b3.astype(f32), 8).reshape(1, 8)

    deg_parts = _deg_kernel(dst1, zp, ones16)

    dis, h1s = pl.pallas_call(
        _tc1_body,
        grid=(GRID,),
        in_specs=[_part_spec(), _row_spec(), _full_spec(128, 128)],
        out_specs=[_row_spec(), _row_spec()],
        out_shape=[jax.ShapeDtypeStruct((NP8, 128), f32),
                   jax.ShapeDtypeStruct((NP8, 128), f32)],
    )(deg_parts, x_p, w1b)

    acc1 = _prop_kernel(h1s.reshape(N_PAD, D), src1, dst1, zp)

    h2s = pl.pallas_call(
        functools.partial(_tc_mid_body, final_scale=False),
        grid=(GRID,),
        in_specs=[_part_spec(), _row_spec(), _row_spec(),
                  _full_spec(128, 128), _full_spec(1, 128)],
        out_specs=_row_spec(),
        out_shape=jax.ShapeDtypeStruct((NP8, 128), f32),
    )(acc1, h1s, dis, w2b, b1t)

    acc2 = _prop_kernel(h2s.reshape(N_PAD, D), src1, dst1, zp)

    t3s = pl.pallas_call(
        functools.partial(_tc_mid_body, final_scale=True),
        grid=(GRID,),
        in_specs=[_part_spec(), _row_spec(), _row_spec(),
                  _full_spec(128, 128), _full_spec(1, 128)],
        out_specs=_row_spec(),
        out_shape=jax.ShapeDtypeStruct((NP8, 128), f32),
    )(acc2, h2s, dis, w2b, b2t)

    acc3 = _prop_kernel(t3s.reshape(N_PAD, D), src1, dst1, zp)

    out8 = pl.pallas_call(
        _tc4_body,
        grid=(GRID,),
        in_specs=[_part_spec(), _row_spec(), _row_spec(),
                  _full_spec(128, 8), _full_spec(1, 8)],
        out_specs=_row_spec(8),
        out_shape=jax.ShapeDtypeStruct((NP8, 8), f32),
    )(acc3, t3s, dis, w3b, b3t)

    return out8.reshape(N_PAD, 1)[:N]
